# Initial kernel scaffold; baseline (speedup 1.0000x reference)
import jax
import jax.numpy as jnp
from jax.experimental import pallas as pl


def kernel(x, indices):
    raise NotImplementedError("write your pallas kernel here")


# speedup vs baseline: 1.3588x; 1.3588x over previous
import functools

import jax
import jax.numpy as jnp
from jax import lax
from jax.experimental import pallas as pl
from jax.experimental.pallas import tpu as pltpu
from jax.experimental.pallas import tpu_sc as plsc

N_BATCH = 256
C_DIM = 768
K_DIM = 120
IN_BLOCK = 12
PER_BATCH_IN = C_DIM * IN_BLOCK
PER_BATCH_OUT = C_DIM * K_DIM
NUM_WORKERS = 32
BATCHES_PER_W = N_BATCH // NUM_WORKERS
PAD = 16


def _make_unpool():
    mesh = plsc.VectorSubcoreMesh(core_axis_name="c", subcore_axis_name="s")

    @functools.partial(
        pl.kernel,
        mesh=mesh,
        compiler_params=pltpu.CompilerParams(needs_layout_passes=False),
        out_type=jax.ShapeDtypeStruct((N_BATCH * PER_BATCH_OUT,), jnp.float32),
        scratch_types=[
            pltpu.VMEM((PER_BATCH_IN + PAD,), jnp.int32),
            pltpu.VMEM((PER_BATCH_IN + PAD,), jnp.float32),
            pltpu.VMEM((PER_BATCH_OUT,), jnp.float32),
        ],
    )
    def unpool(key_hbm, val_hbm, out_hbm, kbuf, vbuf, obuf):
        wid = lax.axis_index("s") * 2 + lax.axis_index("c")
        lane = lax.iota(jnp.int32, 16)
        zeros16 = jnp.zeros((16,), jnp.float32)

        def batch_body(b, carry):
            n = wid * BATCHES_PER_W + b
            in_base = n * PER_BATCH_IN
            pltpu.sync_copy(key_hbm.at[pl.ds(in_base, PER_BATCH_IN + PAD)], kbuf)
            pltpu.sync_copy(val_hbm.at[pl.ds(in_base, PER_BATCH_IN + PAD)], vbuf)

            def zero_body(i, c):
                obuf[pl.ds(i * 16, 16)] = zeros16
                return c

            lax.fori_loop(0, PER_BATCH_OUT // 16, zero_body, 0)

            key_base = n * PER_BATCH_OUT

            def scat_body(i, c):
                base = i * 16
                kv = kbuf[pl.ds(base, 16)]
                knext = plsc.load_gather(kbuf, [lane + (base + 1)])
                vv = vbuf[pl.ds(base, 16)]
                is_last = kv != knext
                rem = kv - key_base
                k = rem // C_DIM
                cc = rem - k * C_DIM
                plsc.store_scatter(obuf, [cc * K_DIM + k], vv, mask=is_last)
                return c

            lax.fori_loop(0, PER_BATCH_IN // 16, scat_body, 0)

            pltpu.sync_copy(obuf, out_hbm.at[pl.ds(n * PER_BATCH_OUT, PER_BATCH_OUT)])
            return carry

        lax.fori_loop(0, BATCHES_PER_W, batch_body, 0)

    return unpool


def kernel(x, indices):
    xf = x.reshape(N_BATCH * C_DIM * IN_BLOCK)
    idf = indices.astype(jnp.int32).reshape(N_BATCH, C_DIM, IN_BLOCK)
    n = jnp.arange(N_BATCH, dtype=jnp.int32)[:, None, None]
    c = jnp.arange(C_DIM, dtype=jnp.int32)[None, :, None]
    gidx = (n * PER_BATCH_OUT + idf * C_DIM + c).reshape(-1)
    sidx, sval = jax.lax.sort((gidx, xf), num_keys=1, is_stable=False, dimension=0)
    sidx = jnp.concatenate([sidx, jnp.full((PAD,), jnp.iinfo(jnp.int32).max, jnp.int32)])
    sval = jnp.concatenate([sval, jnp.zeros((PAD,), jnp.float32)])
    out = _make_unpool()(sidx, sval)
    return out.reshape(N_BATCH, C_DIM, 4, 5, 6)


# speedup vs baseline: 2.3435x; 1.7247x over previous
import functools

import jax
import jax.numpy as jnp
from jax import lax
from jax.experimental import pallas as pl
from jax.experimental.pallas import tpu as pltpu
from jax.experimental.pallas import tpu_sc as plsc

N_BATCH = 256
C_DIM = 768
K_DIM = 120
IN_BLOCK = 12
PER_BATCH_IN = C_DIM * IN_BLOCK
PER_BATCH_OUT = C_DIM * K_DIM
NUM_WORKERS = 32
BATCHES_PER_W = N_BATCH // NUM_WORKERS
PAD = 16


def _make_unpool():
    mesh = plsc.VectorSubcoreMesh(core_axis_name="c", subcore_axis_name="s")

    @functools.partial(
        pl.kernel,
        mesh=mesh,
        compiler_params=pltpu.CompilerParams(needs_layout_passes=False),
        out_type=jax.ShapeDtypeStruct((N_BATCH * PER_BATCH_OUT,), jnp.float32),
        scratch_types=[
            pltpu.VMEM((PER_BATCH_IN + PAD,), jnp.int32),
            pltpu.VMEM((PER_BATCH_IN + PAD,), jnp.float32),
            pltpu.VMEM((PER_BATCH_OUT,), jnp.float32),
        ],
    )
    def unpool(key_hbm, val_hbm, out_hbm, kbuf, vbuf, obuf):
        wid = lax.axis_index("s") * 2 + lax.axis_index("c")
        lane = lax.iota(jnp.int32, 16)
        zeros16 = jnp.zeros((16,), jnp.float32)

        def batch_body(b, carry):
            n = wid * BATCHES_PER_W + b
            in_base = n * PER_BATCH_IN
            pltpu.sync_copy(key_hbm.at[pl.ds(in_base, PER_BATCH_IN + PAD)], kbuf)
            pltpu.sync_copy(val_hbm.at[pl.ds(in_base, PER_BATCH_IN + PAD)], vbuf)

            def zero_body(i, c):
                obuf[pl.ds(i * 16, 16)] = zeros16
                return c

            lax.fori_loop(0, PER_BATCH_OUT // 16, zero_body, 0)

            key_base = n * PER_BATCH_OUT

            def scat_body(i, c):
                base = i * 16
                kv = kbuf[pl.ds(base, 16)]
                knext = plsc.load_gather(kbuf, [lane + (base + 1)])
                vv = vbuf[pl.ds(base, 16)]
                is_last = kv != knext
                plsc.store_scatter(obuf, [kv - key_base], vv, mask=is_last)
                return c

            lax.fori_loop(0, PER_BATCH_IN // 16, scat_body, 0)

            pltpu.sync_copy(obuf, out_hbm.at[pl.ds(n * PER_BATCH_OUT, PER_BATCH_OUT)])
            return carry

        lax.fori_loop(0, BATCHES_PER_W, batch_body, 0)

    return unpool


def kernel(x, indices):
    xf = x.reshape(N_BATCH * C_DIM * IN_BLOCK)
    idx4 = indices.astype(jnp.int32).reshape(N_BATCH, C_DIM, IN_BLOCK, 1)
    idx4 = jnp.where(idx4 < 0, idx4 + K_DIM, idx4)
    n4 = jax.lax.broadcasted_iota(jnp.int32, (N_BATCH, C_DIM, IN_BLOCK, 1), 0)
    c4 = jax.lax.broadcasted_iota(jnp.int32, (N_BATCH, C_DIM, IN_BLOCK, 1), 1)
    cat = jnp.concatenate([n4, c4, idx4], axis=3)
    strides = jnp.array([PER_BATCH_OUT, 1, C_DIM], jnp.int32)
    gidx = jnp.sum(cat * strides, axis=3).reshape(-1)
    sidx, sval = jax.lax.sort((gidx, xf), num_keys=1, is_stable=False, dimension=0)
    sidx = jnp.concatenate([sidx, jnp.full((PAD,), jnp.iinfo(jnp.int32).max, jnp.int32)])
    sval = jnp.concatenate([sval, jnp.zeros((PAD,), jnp.float32)])
    out = _make_unpool()(sidx, sval)
    out = out.reshape(N_BATCH, K_DIM, C_DIM).transpose(0, 2, 1)
    return out.reshape(N_BATCH, C_DIM, 4, 5, 6)


# speedup vs baseline: 4.4584x; 1.9025x over previous
import functools

import jax
import jax.numpy as jnp
from jax import lax
from jax.experimental import pallas as pl
from jax.experimental.pallas import tpu as pltpu
from jax.experimental.pallas import tpu_sc as plsc

N_BATCH = 256
C_DIM = 768
K_DIM = 120
IN_BLOCK = 12
PER_BATCH_IN = C_DIM * IN_BLOCK
PER_BATCH_OUT = C_DIM * K_DIM
NUM_WORKERS = 32
BATCHES_PER_W = N_BATCH // NUM_WORKERS
PAD = 16

_M_TABLE = tuple((j // 6 * 3 + j % 3) * 2 + (j // 3) % 2 for j in range(IN_BLOCK))


def _make_unpool():
    mesh = plsc.VectorSubcoreMesh(core_axis_name="c", subcore_axis_name="s")

    @functools.partial(
        pl.kernel,
        mesh=mesh,
        compiler_params=pltpu.CompilerParams(needs_layout_passes=False),
        out_type=jax.ShapeDtypeStruct((N_BATCH * PER_BATCH_OUT,), jnp.float32),
        scratch_types=[
            pltpu.VMEM((PER_BATCH_IN + PAD,), jnp.int32),
            pltpu.VMEM((PER_BATCH_IN + PAD,), jnp.int32),
            pltpu.VMEM((PER_BATCH_IN,), jnp.float32),
            pltpu.VMEM((16,), jnp.int32),
            pltpu.VMEM((PER_BATCH_OUT,), jnp.float32),
        ],
    )
    def unpool(key_hbm, tag_hbm, xt_hbm, mtab_hbm, out_hbm, kbuf, tbuf, xbuf, mbuf, obuf):
        wid = lax.axis_index("s") * 2 + lax.axis_index("c")
        lane = lax.iota(jnp.int32, 16)
        zeros16 = jnp.zeros((16,), jnp.float32)
        pltpu.sync_copy(mtab_hbm, mbuf)

        def batch_body(b, carry):
            n = wid * BATCHES_PER_W + b
            in_base = n * PER_BATCH_IN
            pltpu.sync_copy(key_hbm.at[pl.ds(in_base, PER_BATCH_IN + PAD)], kbuf)
            pltpu.sync_copy(tag_hbm.at[pl.ds(in_base, PER_BATCH_IN + PAD)], tbuf)
            pltpu.sync_copy(xt_hbm.at[pl.ds(in_base, PER_BATCH_IN)], xbuf)

            def zero_body(i, c):
                obuf[pl.ds(i * 16, 16)] = zeros16
                return c

            lax.fori_loop(0, PER_BATCH_OUT // 16, zero_body, 0)

            key_base = n * PER_BATCH_OUT

            def scat_body(i, c):
                base = i * 16
                kv = kbuf[pl.ds(base, 16)]
                knext = plsc.load_gather(kbuf, [lane + (base + 1)])
                tv = tbuf[pl.ds(base, 16)]
                rel = tv - in_base
                cc = rel // IN_BLOCK
                jrm = rel - cc * IN_BLOCK
                m = plsc.load_gather(mbuf, [jrm])
                xv = plsc.load_gather(xbuf, [m * C_DIM + cc])
                is_last = kv != knext
                plsc.store_scatter(obuf, [kv - key_base], xv, mask=is_last)
                return c

            lax.fori_loop(0, PER_BATCH_IN // 16, scat_body, 0)

            pltpu.sync_copy(obuf, out_hbm.at[pl.ds(n * PER_BATCH_OUT, PER_BATCH_OUT)])
            return carry

        lax.fori_loop(0, BATCHES_PER_W, batch_body, 0)

    return unpool


def kernel(x, indices):
    idx4 = indices.astype(jnp.int32).reshape(N_BATCH, C_DIM, IN_BLOCK, 1)
    idx4 = jnp.where(idx4 < 0, idx4 + K_DIM, idx4)
    n4 = jax.lax.broadcasted_iota(jnp.int32, (N_BATCH, C_DIM, IN_BLOCK, 1), 0)
    c4 = jax.lax.broadcasted_iota(jnp.int32, (N_BATCH, C_DIM, IN_BLOCK, 1), 1)
    cat = jnp.concatenate([n4, c4, idx4], axis=3)
    strides = jnp.array([PER_BATCH_OUT, 1, C_DIM], jnp.int32)
    gidx = jnp.sum(cat * strides, axis=3).reshape(-1)
    tags = jax.lax.iota(jnp.int32, N_BATCH * C_DIM * IN_BLOCK)
    sidx, stag = jax.lax.sort((gidx, tags), num_keys=1, is_stable=False, dimension=0)
    sidx = jnp.concatenate([sidx, jnp.full((PAD,), jnp.iinfo(jnp.int32).max, jnp.int32)])
    stag = jnp.concatenate([stag, jnp.zeros((PAD,), jnp.int32)])
    xt = x.transpose(0, 2, 4, 3, 1).reshape(-1)
    mtab = jnp.array(_M_TABLE + (0,) * (16 - IN_BLOCK), jnp.int32)
    out = _make_unpool()(sidx, stag, xt, mtab)
    out = out.reshape(N_BATCH, K_DIM, C_DIM).transpose(0, 2, 1)
    return out.reshape(N_BATCH, C_DIM, 4, 5, 6)


# speedup vs baseline: 4.4774x; 1.0042x over previous
import functools

import jax
import jax.numpy as jnp
from jax import lax
from jax.experimental import pallas as pl
from jax.experimental.pallas import tpu as pltpu
from jax.experimental.pallas import tpu_sc as plsc

N_BATCH = 256
C_DIM = 768
K_DIM = 120
IN_BLOCK = 12
PER_BATCH_IN = C_DIM * IN_BLOCK
PER_BATCH_OUT = C_DIM * K_DIM
NUM_WORKERS = 32
BATCHES_PER_W = N_BATCH // NUM_WORKERS
PAD = 16

_M_TABLE = tuple((j // 6 * 3 + j % 3) * 2 + (j // 3) % 2 for j in range(IN_BLOCK))


def _make_unpool():
    mesh = plsc.VectorSubcoreMesh(core_axis_name="c", subcore_axis_name="s")

    @functools.partial(
        pl.kernel,
        mesh=mesh,
        compiler_params=pltpu.CompilerParams(needs_layout_passes=False),
        out_type=jax.ShapeDtypeStruct((N_BATCH * PER_BATCH_OUT,), jnp.float32),
        scratch_types=[
            pltpu.VMEM((PER_BATCH_IN + PAD,), jnp.int32),
            pltpu.VMEM((PER_BATCH_IN,), jnp.int32),
            pltpu.VMEM((PER_BATCH_IN,), jnp.float32),
            pltpu.VMEM((16,), jnp.int32),
            pltpu.VMEM((PER_BATCH_OUT,), jnp.float32),
        ],
    )
    def unpool(key_hbm, tag_hbm, xt_hbm, mtab_hbm, out_hbm, kbuf, tbuf, xbuf, mbuf, obuf):
        wid = lax.axis_index("s") * 2 + lax.axis_index("c")
        lane = lax.iota(jnp.int32, 16)
        zeros16 = jnp.zeros((16,), jnp.float32)
        pltpu.sync_copy(mtab_hbm, mbuf)
        kbuf[pl.ds(PER_BATCH_IN, 16)] = jnp.full((16,), jnp.iinfo(jnp.int32).max, jnp.int32)

        def batch_body(b, carry):
            n = wid * BATCHES_PER_W + b
            in_base = n * PER_BATCH_IN
            pltpu.sync_copy(key_hbm.at[pl.ds(in_base, PER_BATCH_IN)], kbuf.at[pl.ds(0, PER_BATCH_IN)])
            pltpu.sync_copy(tag_hbm.at[pl.ds(in_base, PER_BATCH_IN)], tbuf)
            pltpu.sync_copy(xt_hbm.at[pl.ds(in_base, PER_BATCH_IN)], xbuf)

            def zero_body(i, c):
                obuf[pl.ds(i * 16, 16)] = zeros16
                return c

            lax.fori_loop(0, PER_BATCH_OUT // 16, zero_body, 0)

            key_base = n * PER_BATCH_OUT

            def scat_body(i, c):
                base = i * 16
                kv = kbuf[pl.ds(base, 16)]
                knext = plsc.load_gather(kbuf, [lane + (base + 1)])
                tv = tbuf[pl.ds(base, 16)]
                rel = tv - in_base
                cc = rel // IN_BLOCK
                jrm = rel - cc * IN_BLOCK
                m = plsc.load_gather(mbuf, [jrm])
                xv = plsc.load_gather(xbuf, [m * C_DIM + cc])
                is_last = kv != knext
                plsc.store_scatter(obuf, [kv - key_base], xv, mask=is_last)
                return c

            lax.fori_loop(0, PER_BATCH_IN // 16, scat_body, 0)

            pltpu.sync_copy(obuf, out_hbm.at[pl.ds(n * PER_BATCH_OUT, PER_BATCH_OUT)])
            return carry

        lax.fori_loop(0, BATCHES_PER_W, batch_body, 0)

    return unpool


def kernel(x, indices):
    idx4 = indices.astype(jnp.int32).reshape(N_BATCH, C_DIM, IN_BLOCK, 1)
    idx4 = jnp.where(idx4 < 0, idx4 + K_DIM, idx4)
    n4 = jax.lax.broadcasted_iota(jnp.int32, (N_BATCH, C_DIM, IN_BLOCK, 1), 0)
    c4 = jax.lax.broadcasted_iota(jnp.int32, (N_BATCH, C_DIM, IN_BLOCK, 1), 1)
    cat = jnp.concatenate([n4, c4, idx4], axis=3)
    strides = jnp.array([PER_BATCH_OUT, 1, C_DIM], jnp.int32)
    gidx = jnp.sum(cat * strides, axis=3).reshape(-1)
    tags = jax.lax.iota(jnp.int32, N_BATCH * C_DIM * IN_BLOCK)
    sidx, stag = jax.lax.sort((gidx, tags), num_keys=1, is_stable=False, dimension=0)
    xt = x.transpose(0, 2, 4, 3, 1).reshape(-1)
    mtab = jnp.array(_M_TABLE + (0,) * (16 - IN_BLOCK), jnp.int32)
    out = _make_unpool()(sidx, stag, xt, mtab)
    out = out.reshape(N_BATCH, K_DIM, C_DIM).transpose(0, 2, 1)
    return out.reshape(N_BATCH, C_DIM, 4, 5, 6)
